# baseline (speedup 1.0000x reference)
import functools

import jax
import jax.numpy as jnp
from jax import lax
from jax.experimental import pallas as pl
from jax.experimental.pallas import tpu as pltpu
from jax.experimental.pallas import tpu_sc as plsc

N_NODES = 10000
N_EDGES = 320000
D = 128

SC_CORES = 2
SC_SUBCORES = 16
NW = SC_CORES * SC_SUBCORES
GROUP_EPW = (1000, 3000, 3000, 3000)
GROUP_SIZES = tuple(NW * e for e in GROUP_EPW)
GROUP_STARTS = tuple(sum(GROUP_SIZES[:g]) for g in range(len(GROUP_SIZES)))
NSPLIT = len(GROUP_EPW)
GCHUNK = 200
SCHUNK = 40


def _f32_dot(x, w):
    return jax.lax.dot_general(x, w, (((1,), (0,)), ((), ())),
                               preferred_element_type=jnp.float32)


def _precompute_body(hn, a, b, pa, pb):
    x = hn[...]
    pa[...] = _f32_dot(x, a[...])
    pb[...] = _f32_dot(x, b[...])


def _precompute(h_node, a, b):
    R = 2000
    grid = (N_NODES // R,)
    return pl.pallas_call(
        _precompute_body,
        grid=grid,
        in_specs=[
            pl.BlockSpec((R, D), lambda i: (i, 0)),
            pl.BlockSpec((D, D), lambda i: (0, 0)),
            pl.BlockSpec((D, D), lambda i: (0, 0)),
        ],
        out_specs=[
            pl.BlockSpec((R, D), lambda i: (i, 0)),
            pl.BlockSpec((R, D), lambda i: (i, 0)),
        ],
        out_shape=[
            jax.ShapeDtypeStruct((N_NODES, D), jnp.float32),
            jax.ShapeDtypeStruct((N_NODES, D), jnp.float32),
        ],
    )(h_node, a, b)


def _edge_mlp_body(gsum, he, cw, w2, b1, b2, lns, lnb, out):
    h = he[...]
    x = gsum[...] + _f32_dot(h, cw[...]) + b1[...]
    x = jnp.maximum(x, 0.0)
    e = _f32_dot(x, w2[...]) + b2[...]
    mu = jnp.mean(e, axis=-1, keepdims=True)
    var = jnp.mean((e - mu) ** 2, axis=-1, keepdims=True)
    e = (e - mu) * jax.lax.rsqrt(var + 1e-5) * lns[...] + lnb[...]
    out[...] = h + e


def _edge_mlp(gsum, h_edge, cw, w2, b1, b2, lns, lnb):
    R = 2000
    eg = gsum.shape[0]
    grid = (eg // R,)
    row = lambda i: (i, 0)
    full = lambda i: (0, 0)
    return pl.pallas_call(
        _edge_mlp_body,
        grid=grid,
        in_specs=[
            pl.BlockSpec((R, D), row),
            pl.BlockSpec((R, D), row),
            pl.BlockSpec((D, D), full),
            pl.BlockSpec((D, D), full),
            pl.BlockSpec((1, D), full),
            pl.BlockSpec((1, D), full),
            pl.BlockSpec((1, D), full),
            pl.BlockSpec((1, D), full),
        ],
        out_specs=pl.BlockSpec((R, D), row),
        out_shape=jax.ShapeDtypeStruct((eg, D), jnp.float32),
    )(gsum, h_edge, cw, w2, b1, b2, lns, lnb)


def _node_mlp_body(hn, p0, p1, wa, wb, w2, b1, b2, lns, lnb, out):
    h = hn[...]
    agg = p0[...] + p1[...]
    x = _f32_dot(h, wa[...]) + _f32_dot(agg, wb[...]) + b1[...]
    x = jnp.maximum(x, 0.0)
    n = _f32_dot(x, w2[...]) + b2[...]
    mu = jnp.mean(n, axis=-1, keepdims=True)
    var = jnp.mean((n - mu) ** 2, axis=-1, keepdims=True)
    n = (n - mu) * jax.lax.rsqrt(var + 1e-5) * lns[...] + lnb[...]
    out[...] = h + n


def _node_mlp(h_node, parts, wa, wb, w2, b1, b2, lns, lnb):
    R = 2000
    nb = N_NODES // R
    grid = (nb,)
    row = lambda i: (i, 0)
    full = lambda i: (0, 0)
    return pl.pallas_call(
        _node_mlp_body,
        grid=grid,
        in_specs=[
            pl.BlockSpec((R, D), row),
            pl.BlockSpec((R, D), row),
            pl.BlockSpec((R, D), lambda i, _nb=nb: (i + _nb, 0)),
            pl.BlockSpec((D, D), full),
            pl.BlockSpec((D, D), full),
            pl.BlockSpec((D, D), full),
            pl.BlockSpec((1, D), full),
            pl.BlockSpec((1, D), full),
            pl.BlockSpec((1, D), full),
            pl.BlockSpec((1, D), full),
        ],
        out_specs=pl.BlockSpec((R, D), row),
        out_shape=jax.ShapeDtypeStruct((N_NODES, D), jnp.float32),
    )(h_node, parts, parts, wa, wb, w2, b1, b2, lns, lnb)


def _sc_mesh():
    return plsc.VectorSubcoreMesh(
        core_axis_name="c", subcore_axis_name="s",
        num_cores=SC_CORES, num_subcores=SC_SUBCORES)


def _sc_gather(pa, pb, src, dst, epw):
    C2 = GCHUNK
    NC2 = epw // C2

    @functools.partial(
        pl.kernel,
        out_type=jax.ShapeDtypeStruct((NW * epw, D), jnp.float32),
        mesh=_sc_mesh(),
        scratch_types=[
            pltpu.VMEM((epw,), jnp.int32),
            pltpu.VMEM((epw,), jnp.int32),
            pltpu.VMEM((C2, D), jnp.float32),
            pltpu.VMEM((C2, D), jnp.float32),
            pltpu.VMEM((C2, D), jnp.float32),
            pltpu.VMEM((C2, D), jnp.float32),
            pltpu.SemaphoreType.DMA,
            pltpu.SemaphoreType.DMA,
            pltpu.SemaphoreType.DMA,
            pltpu.SemaphoreType.DMA,
        ],
    )
    def k(pa_hbm, pb_hbm, src_hbm, dst_hbm, gsum_hbm,
          idx_s, idx_d, rs0, rd0, rs1, rd1, sg0, sg1, sw0, sw1):
        wid = lax.axis_index("s") * SC_CORES + lax.axis_index("c")
        base = wid * epw
        pltpu.sync_copy(src_hbm.at[pl.ds(base, epw)], idx_s)
        pltpu.sync_copy(dst_hbm.at[pl.ds(base, epw)], idx_d)

        slots = ((rs0, rd0, sg0, sw0), (rs1, rd1, sg1, sw1))

        def g_copies(chunk, slot):
            rs, rd, sg, _ = slots[slot]
            o = pl.multiple_of(chunk * C2, 8)
            return (pltpu.make_async_copy(
                        pa_hbm.at[idx_s.at[pl.ds(o, C2)]], rs, sg),
                    pltpu.make_async_copy(
                        pb_hbm.at[idx_d.at[pl.ds(o, C2)]], rd, sg))

        def w_copy(chunk, slot):
            rs, _, _, sw = slots[slot]
            o = pl.multiple_of(base + chunk * C2, 8)
            return pltpu.make_async_copy(rs, gsum_hbm.at[pl.ds(o, C2)], sw)

        def start_g(chunk, slot):
            for cp in g_copies(chunk, slot):
                cp.start()

        def wait_g(chunk, slot):
            for cp in g_copies(chunk, slot):
                cp.wait()

        def vsum(slot):
            rs, rd, _, _ = slots[slot]

            @plsc.parallel_loop(0, C2, 1, unroll=2)
            def _add(r):
                for cc in range(D // 16):
                    sl = pl.ds(cc * 16, 16)
                    rs[r, sl] = rs[r, sl] + rd[r, sl]

        start_g(0, 0)
        start_g(1, 1)

        npairs = NC2 // 2

        def body(j, _):
            a = 2 * j
            for slot in (0, 1):
                c = a + slot
                wait_g(c, slot)
                vsum(slot)
                w_copy(c, slot).start()

            @pl.when(j < npairs - 1)
            def _next():
                for slot in (0, 1):
                    c = a + slot
                    w_copy(c, slot).wait()
                    start_g(c + 2, slot)

            return ()

        lax.fori_loop(0, npairs, body, ())
        if NC2 % 2:
            last = NC2 - 1
            w_copy(last - 2, 0).wait()
            start_g(last, 0)
            wait_g(last, 0)
            vsum(0)
            w_copy(last, 0).start()
            w_copy(last - 1, 1).wait()
            w_copy(last, 0).wait()
        else:
            w_copy(NC2 - 2, 0).wait()
            w_copy(NC2 - 1, 1).wait()

    return k(pa, pb, src, dst)


def _sc_segsum(h_edge, dst3, init, epw):
    nchunk = epw // SCHUNK

    @functools.partial(
        pl.kernel,
        out_type=jax.ShapeDtypeStruct((2 * N_NODES, D), jnp.float32),
        mesh=_sc_mesh(),
        scratch_types=[
            pltpu.VMEM((nchunk, SCHUNK), jnp.int32),
            pltpu.VMEM((SCHUNK, D), jnp.float32),
            pltpu.VMEM((SCHUNK, D), jnp.float32),
            pltpu.VMEM_SHARED((N_NODES, D), jnp.float32),
            pltpu.SemaphoreType.DMA,
            pltpu.SemaphoreType.DMA,
        ],
    )
    def k(he_hbm, dst_hbm, init_hbm, out_hbm, idx_v, r0, r1, shared,
          sl0, sl1):
        c = lax.axis_index("c")
        s = lax.axis_index("s")
        wid = c * SC_SUBCORES + s
        base = wid * epw

        @pl.when(s == 0)
        def _load_init():
            pltpu.sync_copy(init_hbm.at[pl.ds(c * N_NODES, N_NODES)], shared)

        pltpu.sync_copy(dst_hbm.at[wid], idx_v)
        plsc.subcore_barrier()

        slots = ((r0, sl0), (r1, sl1))

        def load_copy(chunk, slot):
            rv, sl = slots[slot]
            off = pl.multiple_of(base + chunk * SCHUNK, 8)
            return pltpu.make_async_copy(he_hbm.at[pl.ds(off, SCHUNK)], rv, sl)

        def process(chunk, slot):
            rv, _ = slots[slot]
            load_copy(chunk, slot).wait()
            pltpu.sync_copy(rv, shared.at[idx_v.at[chunk]], add=True)

        load_copy(0, 0).start()
        load_copy(1, 1).start()

        def body(j, _):
            a = 2 * j
            for slot in (0, 1):
                process(a + slot, slot)

            @pl.when(j < nchunk // 2 - 1)
            def _next():
                for slot in (0, 1):
                    load_copy(a + slot + 2, slot).start()

            return ()

        lax.fori_loop(0, nchunk // 2, body, ())
        if nchunk % 2:
            load_copy(nchunk - 1, 0).start()
            process(nchunk - 1, 0)

        plsc.subcore_barrier()

        rpt = 624
        r0 = s * rpt
        pltpu.sync_copy(shared.at[pl.ds(r0, rpt)],
                        out_hbm.at[pl.ds(c * N_NODES + r0, rpt)])

        @pl.when(s == 0)
        def _tail():
            t0 = SC_SUBCORES * rpt
            pltpu.sync_copy(shared.at[pl.ds(t0, N_NODES - t0)],
                            out_hbm.at[pl.ds(c * N_NODES + t0, N_NODES - t0)])

    return k(h_edge, dst3, init)


def _gslice(x, g):
    return x[GROUP_STARTS[g]:GROUP_STARTS[g] + GROUP_SIZES[g]]


def kernel(h_node, edge_index, h_edge, ew1, eb1, ew2, eb2, eln_s, eln_b,
           nw1, nb1, nw2, nb2, nln_s, nln_b):
    num_convs = ew1.shape[0]
    src = edge_index[0].astype(jnp.int32)
    dst = edge_index[1].astype(jnp.int32)
    src_g = [_gslice(src, g) for g in range(NSPLIT)]
    dst_g = [_gslice(dst, g) for g in range(NSPLIT)]
    dst3_g = [dst_g[g].reshape(NW, GROUP_EPW[g] // SCHUNK, SCHUNK)
              for g in range(NSPLIT)]
    he_g = [_gslice(h_edge, g) for g in range(NSPLIT)]
    zeros2 = jnp.zeros((2 * N_NODES, D), jnp.float32)

    r1 = lambda v: v.reshape(1, D)

    for i in range(num_convs):
        a = ew1[i, :D]
        b = ew1[i, D:2 * D]
        cw = ew1[i, 2 * D:]
        pa, pb = _precompute(h_node, a, b)
        gg = [_sc_gather(pa, pb, src_g[g], dst_g[g], GROUP_EPW[g])
              for g in range(NSPLIT)]
        he_g = [_edge_mlp(gg[g], he_g[g], cw, ew2[i],
                          r1(eb1[i]), r1(eb2[i]), r1(eln_s[i]), r1(eln_b[i]))
                for g in range(NSPLIT)]
        parts = zeros2
        for g in range(NSPLIT):
            parts = _sc_segsum(he_g[g], dst3_g[g], parts, GROUP_EPW[g])
        h_node = _node_mlp(h_node, parts, nw1[i, :D], nw1[i, D:], nw2[i],
                           r1(nb1[i]), r1(nb2[i]), r1(nln_s[i]), r1(nln_b[i]))
    return (h_node, jnp.concatenate(he_g, axis=0))


# speedup vs baseline: 1.0467x; 1.0467x over previous
import functools

import jax
import jax.numpy as jnp
from jax import lax
from jax.experimental import pallas as pl
from jax.experimental.pallas import tpu as pltpu
from jax.experimental.pallas import tpu_sc as plsc

N_NODES = 10000
N_EDGES = 320000
D = 128

SC_CORES = 2
SC_SUBCORES = 16
NW = SC_CORES * SC_SUBCORES
GROUP_EPW = (5000, 5000)
GROUP_SIZES = tuple(NW * e for e in GROUP_EPW)
GROUP_STARTS = tuple(sum(GROUP_SIZES[:g]) for g in range(len(GROUP_SIZES)))
NSPLIT = len(GROUP_EPW)
GCHUNK = 200
SCHUNK = 40


def _f32_dot(x, w):
    return jax.lax.dot_general(x, w, (((1,), (0,)), ((), ())),
                               preferred_element_type=jnp.float32)


def _precompute_body(hn, a, b, pa, pb):
    x = hn[...]
    pa[...] = _f32_dot(x, a[...])
    pb[...] = _f32_dot(x, b[...])


def _precompute(h_node, a, b):
    R = 2000
    grid = (N_NODES // R,)
    return pl.pallas_call(
        _precompute_body,
        grid=grid,
        in_specs=[
            pl.BlockSpec((R, D), lambda i: (i, 0)),
            pl.BlockSpec((D, D), lambda i: (0, 0)),
            pl.BlockSpec((D, D), lambda i: (0, 0)),
        ],
        out_specs=[
            pl.BlockSpec((R, D), lambda i: (i, 0)),
            pl.BlockSpec((R, D), lambda i: (i, 0)),
        ],
        out_shape=[
            jax.ShapeDtypeStruct((N_NODES, D), jnp.float32),
            jax.ShapeDtypeStruct((N_NODES, D), jnp.float32),
        ],
    )(h_node, a, b)


def _edge_mlp_body(gsum, he, cw, w2, b1, b2, lns, lnb, out):
    h = he[...]
    x = gsum[...] + _f32_dot(h, cw[...]) + b1[...]
    x = jnp.maximum(x, 0.0)
    e = _f32_dot(x, w2[...]) + b2[...]
    mu = jnp.mean(e, axis=-1, keepdims=True)
    var = jnp.mean((e - mu) ** 2, axis=-1, keepdims=True)
    e = (e - mu) * jax.lax.rsqrt(var + 1e-5) * lns[...] + lnb[...]
    out[...] = h + e


def _edge_mlp(gsum, h_edge, cw, w2, b1, b2, lns, lnb):
    R = 2000
    eg = gsum.shape[0]
    grid = (eg // R,)
    row = lambda i: (i, 0)
    full = lambda i: (0, 0)
    return pl.pallas_call(
        _edge_mlp_body,
        grid=grid,
        in_specs=[
            pl.BlockSpec((R, D), row),
            pl.BlockSpec((R, D), row),
            pl.BlockSpec((D, D), full),
            pl.BlockSpec((D, D), full),
            pl.BlockSpec((1, D), full),
            pl.BlockSpec((1, D), full),
            pl.BlockSpec((1, D), full),
            pl.BlockSpec((1, D), full),
        ],
        out_specs=pl.BlockSpec((R, D), row),
        out_shape=jax.ShapeDtypeStruct((eg, D), jnp.float32),
    )(gsum, h_edge, cw, w2, b1, b2, lns, lnb)


def _node_mlp_body(hn, p0, p1, wa, wb, w2, b1, b2, lns, lnb, out):
    h = hn[...]
    agg = p0[...] + p1[...]
    x = _f32_dot(h, wa[...]) + _f32_dot(agg, wb[...]) + b1[...]
    x = jnp.maximum(x, 0.0)
    n = _f32_dot(x, w2[...]) + b2[...]
    mu = jnp.mean(n, axis=-1, keepdims=True)
    var = jnp.mean((n - mu) ** 2, axis=-1, keepdims=True)
    n = (n - mu) * jax.lax.rsqrt(var + 1e-5) * lns[...] + lnb[...]
    out[...] = h + n


def _node_mlp(h_node, parts, wa, wb, w2, b1, b2, lns, lnb):
    R = 2000
    nb = N_NODES // R
    grid = (nb,)
    row = lambda i: (i, 0)
    full = lambda i: (0, 0)
    return pl.pallas_call(
        _node_mlp_body,
        grid=grid,
        in_specs=[
            pl.BlockSpec((R, D), row),
            pl.BlockSpec((R, D), row),
            pl.BlockSpec((R, D), lambda i, _nb=nb: (i + _nb, 0)),
            pl.BlockSpec((D, D), full),
            pl.BlockSpec((D, D), full),
            pl.BlockSpec((D, D), full),
            pl.BlockSpec((1, D), full),
            pl.BlockSpec((1, D), full),
            pl.BlockSpec((1, D), full),
            pl.BlockSpec((1, D), full),
        ],
        out_specs=pl.BlockSpec((R, D), row),
        out_shape=jax.ShapeDtypeStruct((N_NODES, D), jnp.float32),
    )(h_node, parts, parts, wa, wb, w2, b1, b2, lns, lnb)


def _sc_mesh():
    return plsc.VectorSubcoreMesh(
        core_axis_name="c", subcore_axis_name="s",
        num_cores=SC_CORES, num_subcores=SC_SUBCORES)


def _sc_gather(pa, pb, src, dst, epw):
    C2 = GCHUNK
    NC2 = epw // C2

    @functools.partial(
        pl.kernel,
        out_type=jax.ShapeDtypeStruct((NW * epw, D), jnp.float32),
        mesh=_sc_mesh(),
        scratch_types=[
            pltpu.VMEM((epw,), jnp.int32),
            pltpu.VMEM((epw,), jnp.int32),
            pltpu.VMEM((C2, D), jnp.float32),
            pltpu.VMEM((C2, D), jnp.float32),
            pltpu.VMEM((C2, D), jnp.float32),
            pltpu.VMEM((C2, D), jnp.float32),
            pltpu.SemaphoreType.DMA,
            pltpu.SemaphoreType.DMA,
            pltpu.SemaphoreType.DMA,
            pltpu.SemaphoreType.DMA,
        ],
    )
    def k(pa_hbm, pb_hbm, src_hbm, dst_hbm, gsum_hbm,
          idx_s, idx_d, rs0, rd0, rs1, rd1, sg0, sg1, sw0, sw1):
        wid = lax.axis_index("s") * SC_CORES + lax.axis_index("c")
        base = wid * epw
        pltpu.sync_copy(src_hbm.at[pl.ds(base, epw)], idx_s)
        pltpu.sync_copy(dst_hbm.at[pl.ds(base, epw)], idx_d)

        slots = ((rs0, rd0, sg0, sw0), (rs1, rd1, sg1, sw1))

        def g_copies(chunk, slot):
            rs, rd, sg, _ = slots[slot]
            o = pl.multiple_of(chunk * C2, 8)
            return (pltpu.make_async_copy(
                        pa_hbm.at[idx_s.at[pl.ds(o, C2)]], rs, sg),
                    pltpu.make_async_copy(
                        pb_hbm.at[idx_d.at[pl.ds(o, C2)]], rd, sg))

        def w_copy(chunk, slot):
            rs, _, _, sw = slots[slot]
            o = pl.multiple_of(base + chunk * C2, 8)
            return pltpu.make_async_copy(rs, gsum_hbm.at[pl.ds(o, C2)], sw)

        def start_g(chunk, slot):
            for cp in g_copies(chunk, slot):
                cp.start()

        def wait_g(chunk, slot):
            for cp in g_copies(chunk, slot):
                cp.wait()

        def vsum(slot):
            rs, rd, _, _ = slots[slot]

            @plsc.parallel_loop(0, C2, 1, unroll=2)
            def _add(r):
                for cc in range(D // 16):
                    sl = pl.ds(cc * 16, 16)
                    rs[r, sl] = rs[r, sl] + rd[r, sl]

        start_g(0, 0)
        start_g(1, 1)

        npairs = NC2 // 2

        def body(j, _):
            a = 2 * j
            for slot in (0, 1):
                c = a + slot
                wait_g(c, slot)
                vsum(slot)
                w_copy(c, slot).start()

            @pl.when(j < npairs - 1)
            def _next():
                for slot in (0, 1):
                    c = a + slot
                    w_copy(c, slot).wait()
                    start_g(c + 2, slot)

            return ()

        lax.fori_loop(0, npairs, body, ())
        if NC2 % 2:
            last = NC2 - 1
            w_copy(last - 2, 0).wait()
            start_g(last, 0)
            wait_g(last, 0)
            vsum(0)
            w_copy(last, 0).start()
            w_copy(last - 1, 1).wait()
            w_copy(last, 0).wait()
        else:
            w_copy(NC2 - 2, 0).wait()
            w_copy(NC2 - 1, 1).wait()

    return k(pa, pb, src, dst)


def _sc_segsum(h_edge, dst3, init, epw):
    nchunk = epw // SCHUNK

    @functools.partial(
        pl.kernel,
        out_type=jax.ShapeDtypeStruct((2 * N_NODES, D), jnp.float32),
        mesh=_sc_mesh(),
        scratch_types=[
            pltpu.VMEM((nchunk, SCHUNK), jnp.int32),
            pltpu.VMEM((SCHUNK, D), jnp.float32),
            pltpu.VMEM((SCHUNK, D), jnp.float32),
            pltpu.VMEM_SHARED((N_NODES, D), jnp.float32),
            pltpu.SemaphoreType.DMA,
            pltpu.SemaphoreType.DMA,
        ],
    )
    def k(he_hbm, dst_hbm, init_hbm, out_hbm, idx_v, r0, r1, shared,
          sl0, sl1):
        c = lax.axis_index("c")
        s = lax.axis_index("s")
        wid = c * SC_SUBCORES + s
        base = wid * epw

        @pl.when(s == 0)
        def _load_init():
            pltpu.sync_copy(init_hbm.at[pl.ds(c * N_NODES, N_NODES)], shared)

        pltpu.sync_copy(dst_hbm.at[wid], idx_v)
        plsc.subcore_barrier()

        slots = ((r0, sl0), (r1, sl1))

        def load_copy(chunk, slot):
            rv, sl = slots[slot]
            off = pl.multiple_of(base + chunk * SCHUNK, 8)
            return pltpu.make_async_copy(he_hbm.at[pl.ds(off, SCHUNK)], rv, sl)

        def process(chunk, slot):
            rv, _ = slots[slot]
            load_copy(chunk, slot).wait()
            pltpu.sync_copy(rv, shared.at[idx_v.at[chunk]], add=True)

        load_copy(0, 0).start()
        load_copy(1, 1).start()

        def body(j, _):
            a = 2 * j
            for slot in (0, 1):
                process(a + slot, slot)

            @pl.when(j < nchunk // 2 - 1)
            def _next():
                for slot in (0, 1):
                    load_copy(a + slot + 2, slot).start()

            return ()

        lax.fori_loop(0, nchunk // 2, body, ())
        if nchunk % 2:
            load_copy(nchunk - 1, 0).start()
            process(nchunk - 1, 0)

        plsc.subcore_barrier()

        rpt = 624
        r0 = s * rpt
        pltpu.sync_copy(shared.at[pl.ds(r0, rpt)],
                        out_hbm.at[pl.ds(c * N_NODES + r0, rpt)])

        @pl.when(s == 0)
        def _tail():
            t0 = SC_SUBCORES * rpt
            pltpu.sync_copy(shared.at[pl.ds(t0, N_NODES - t0)],
                            out_hbm.at[pl.ds(c * N_NODES + t0, N_NODES - t0)])

    return k(h_edge, dst3, init)


def _gslice(x, g):
    return x[GROUP_STARTS[g]:GROUP_STARTS[g] + GROUP_SIZES[g]]


def kernel(h_node, edge_index, h_edge, ew1, eb1, ew2, eb2, eln_s, eln_b,
           nw1, nb1, nw2, nb2, nln_s, nln_b):
    num_convs = ew1.shape[0]
    src = edge_index[0].astype(jnp.int32)
    dst = edge_index[1].astype(jnp.int32)
    src_g = [_gslice(src, g) for g in range(NSPLIT)]
    dst_g = [_gslice(dst, g) for g in range(NSPLIT)]
    dst3_g = [dst_g[g].reshape(NW, GROUP_EPW[g] // SCHUNK, SCHUNK)
              for g in range(NSPLIT)]
    he_g = [_gslice(h_edge, g) for g in range(NSPLIT)]
    zeros2 = jnp.zeros((2 * N_NODES, D), jnp.float32)

    r1 = lambda v: v.reshape(1, D)

    for i in range(num_convs):
        a = ew1[i, :D]
        b = ew1[i, D:2 * D]
        cw = ew1[i, 2 * D:]
        pa, pb = _precompute(h_node, a, b)
        gg = [_sc_gather(pa, pb, src_g[g], dst_g[g], GROUP_EPW[g])
              for g in range(NSPLIT)]
        he_g = [_edge_mlp(gg[g], he_g[g], cw, ew2[i],
                          r1(eb1[i]), r1(eb2[i]), r1(eln_s[i]), r1(eln_b[i]))
                for g in range(NSPLIT)]
        parts = zeros2
        for g in range(NSPLIT):
            parts = _sc_segsum(he_g[g], dst3_g[g], parts, GROUP_EPW[g])
        h_node = _node_mlp(h_node, parts, nw1[i, :D], nw1[i, D:], nw2[i],
                           r1(nb1[i]), r1(nb2[i]), r1(nln_s[i]), r1(nln_b[i]))
    return (h_node, jnp.concatenate(he_g, axis=0))


# speedup vs baseline: 1.0616x; 1.0143x over previous
import functools

import jax
import jax.numpy as jnp
from jax import lax
from jax.experimental import pallas as pl
from jax.experimental.pallas import tpu as pltpu
from jax.experimental.pallas import tpu_sc as plsc

N_NODES = 10000
N_EDGES = 320000
D = 128

SC_CORES = 2
SC_SUBCORES = 16
NW = SC_CORES * SC_SUBCORES
GROUP_EPW = (5000, 5000)
GROUP_SIZES = tuple(NW * e for e in GROUP_EPW)
GROUP_STARTS = tuple(sum(GROUP_SIZES[:g]) for g in range(len(GROUP_SIZES)))
NSPLIT = len(GROUP_EPW)
GCHUNK = 200
SCHUNK = 40


def _f32_dot(x, w):
    return jax.lax.dot_general(x, w, (((1,), (0,)), ((), ())),
                               preferred_element_type=jnp.float32)


def _precompute_body(hn, a, b, pa, pb):
    x = hn[...]
    pa[...] = _f32_dot(x, a[...])
    pb[...] = _f32_dot(x, b[...])


def _precompute(h_node, a, b):
    R = 2000
    grid = (N_NODES // R,)
    return pl.pallas_call(
        _precompute_body,
        grid=grid,
        in_specs=[
            pl.BlockSpec((R, D), lambda i: (i, 0)),
            pl.BlockSpec((D, D), lambda i: (0, 0)),
            pl.BlockSpec((D, D), lambda i: (0, 0)),
        ],
        out_specs=[
            pl.BlockSpec((R, D), lambda i: (i, 0)),
            pl.BlockSpec((R, D), lambda i: (i, 0)),
        ],
        out_shape=[
            jax.ShapeDtypeStruct((N_NODES, D), jnp.float32),
            jax.ShapeDtypeStruct((N_NODES, D), jnp.float32),
        ],
    )(h_node, a, b)


def _edge_mlp_body(gsum, he, cw, w2, b1, b2, lns, lnb, out):
    h = he[...]
    x = gsum[...] + _f32_dot(h, cw[...]) + b1[...]
    x = jnp.maximum(x, 0.0)
    e = _f32_dot(x, w2[...]) + b2[...]
    mu = jnp.mean(e, axis=-1, keepdims=True)
    var = jnp.mean((e - mu) ** 2, axis=-1, keepdims=True)
    e = (e - mu) * jax.lax.rsqrt(var + 1e-5) * lns[...] + lnb[...]
    out[...] = h + e


def _edge_mlp(gsum, h_edge, cw, w2, b1, b2, lns, lnb):
    R = 2000
    eg = gsum.shape[0]
    grid = (eg // R,)
    row = lambda i: (i, 0)
    full = lambda i: (0, 0)
    return pl.pallas_call(
        _edge_mlp_body,
        grid=grid,
        in_specs=[
            pl.BlockSpec((R, D), row),
            pl.BlockSpec((R, D), row),
            pl.BlockSpec((D, D), full),
            pl.BlockSpec((D, D), full),
            pl.BlockSpec((1, D), full),
            pl.BlockSpec((1, D), full),
            pl.BlockSpec((1, D), full),
            pl.BlockSpec((1, D), full),
        ],
        out_specs=pl.BlockSpec((R, D), row),
        out_shape=jax.ShapeDtypeStruct((eg, D), jnp.float32),
    )(gsum, h_edge, cw, w2, b1, b2, lns, lnb)


def _node_new(hn, p0, p1, wa, wb, w2, b1, b2, lns, lnb):
    h = hn[...]
    agg = p0[...] + p1[...]
    x = _f32_dot(h, wa[...]) + _f32_dot(agg, wb[...]) + b1[...]
    x = jnp.maximum(x, 0.0)
    n = _f32_dot(x, w2[...]) + b2[...]
    mu = jnp.mean(n, axis=-1, keepdims=True)
    var = jnp.mean((n - mu) ** 2, axis=-1, keepdims=True)
    n = (n - mu) * jax.lax.rsqrt(var + 1e-5) * lns[...] + lnb[...]
    return h + n


def _node_mlp_body(hn, p0, p1, wa, wb, w2, b1, b2, lns, lnb, out):
    out[...] = _node_new(hn, p0, p1, wa, wb, w2, b1, b2, lns, lnb)


def _node_mlp_fused_body(hn, p0, p1, wa, wb, w2, b1, b2, lns, lnb,
                         na, nb2_, out, pa, pb):
    hnew = _node_new(hn, p0, p1, wa, wb, w2, b1, b2, lns, lnb)
    out[...] = hnew
    pa[...] = _f32_dot(hnew, na[...])
    pb[...] = _f32_dot(hnew, nb2_[...])


def _node_mlp(h_node, parts, wa, wb, w2, b1, b2, lns, lnb,
              next_a=None, next_b=None):
    R = 2000
    nb = N_NODES // R
    grid = (nb,)
    row = lambda i: (i, 0)
    full = lambda i: (0, 0)
    fused = next_a is not None
    in_specs = [
        pl.BlockSpec((R, D), row),
        pl.BlockSpec((R, D), row),
        pl.BlockSpec((R, D), lambda i, _nb=nb: (i + _nb, 0)),
        pl.BlockSpec((D, D), full),
        pl.BlockSpec((D, D), full),
        pl.BlockSpec((D, D), full),
        pl.BlockSpec((1, D), full),
        pl.BlockSpec((1, D), full),
        pl.BlockSpec((1, D), full),
        pl.BlockSpec((1, D), full),
    ]
    args = [h_node, parts, parts, wa, wb, w2, b1, b2, lns, lnb]
    out_spec = pl.BlockSpec((R, D), row)
    out_shape = jax.ShapeDtypeStruct((N_NODES, D), jnp.float32)
    if fused:
        in_specs += [pl.BlockSpec((D, D), full), pl.BlockSpec((D, D), full)]
        args += [next_a, next_b]
        return pl.pallas_call(
            _node_mlp_fused_body,
            grid=grid,
            in_specs=in_specs,
            out_specs=[out_spec, out_spec, out_spec],
            out_shape=[out_shape, out_shape, out_shape],
        )(*args)
    return pl.pallas_call(
        _node_mlp_body,
        grid=grid,
        in_specs=in_specs,
        out_specs=out_spec,
        out_shape=out_shape,
    )(*args)


def _sc_mesh():
    return plsc.VectorSubcoreMesh(
        core_axis_name="c", subcore_axis_name="s",
        num_cores=SC_CORES, num_subcores=SC_SUBCORES)


def _sc_gather(pa, pb, src, dst, epw):
    C2 = GCHUNK
    NC2 = epw // C2

    @functools.partial(
        pl.kernel,
        out_type=jax.ShapeDtypeStruct((NW * epw, D), jnp.float32),
        mesh=_sc_mesh(),
        scratch_types=[
            pltpu.VMEM((epw,), jnp.int32),
            pltpu.VMEM((epw,), jnp.int32),
            pltpu.VMEM((C2, D), jnp.float32),
            pltpu.VMEM((C2, D), jnp.float32),
            pltpu.VMEM((C2, D), jnp.float32),
            pltpu.VMEM((C2, D), jnp.float32),
            pltpu.SemaphoreType.DMA,
            pltpu.SemaphoreType.DMA,
            pltpu.SemaphoreType.DMA,
            pltpu.SemaphoreType.DMA,
        ],
    )
    def k(pa_hbm, pb_hbm, src_hbm, dst_hbm, gsum_hbm,
          idx_s, idx_d, rs0, rd0, rs1, rd1, sg0, sg1, sw0, sw1):
        wid = lax.axis_index("s") * SC_CORES + lax.axis_index("c")
        base = wid * epw
        pltpu.sync_copy(src_hbm.at[pl.ds(base, epw)], idx_s)
        pltpu.sync_copy(dst_hbm.at[pl.ds(base, epw)], idx_d)

        slots = ((rs0, rd0, sg0, sw0), (rs1, rd1, sg1, sw1))

        def g_copies(chunk, slot):
            rs, rd, sg, _ = slots[slot]
            o = pl.multiple_of(chunk * C2, 8)
            return (pltpu.make_async_copy(
                        pa_hbm.at[idx_s.at[pl.ds(o, C2)]], rs, sg),
                    pltpu.make_async_copy(
                        pb_hbm.at[idx_d.at[pl.ds(o, C2)]], rd, sg))

        def w_copy(chunk, slot):
            rs, _, _, sw = slots[slot]
            o = pl.multiple_of(base + chunk * C2, 8)
            return pltpu.make_async_copy(rs, gsum_hbm.at[pl.ds(o, C2)], sw)

        def start_g(chunk, slot):
            for cp in g_copies(chunk, slot):
                cp.start()

        def wait_g(chunk, slot):
            for cp in g_copies(chunk, slot):
                cp.wait()

        def vsum(slot):
            rs, rd, _, _ = slots[slot]

            @plsc.parallel_loop(0, C2, 1, unroll=2)
            def _add(r):
                for cc in range(D // 16):
                    sl = pl.ds(cc * 16, 16)
                    rs[r, sl] = rs[r, sl] + rd[r, sl]

        start_g(0, 0)
        start_g(1, 1)

        npairs = NC2 // 2

        def body(j, _):
            a = 2 * j
            for slot in (0, 1):
                c = a + slot
                wait_g(c, slot)
                vsum(slot)
                w_copy(c, slot).start()

            @pl.when(j < npairs - 1)
            def _next():
                for slot in (0, 1):
                    c = a + slot
                    w_copy(c, slot).wait()
                    start_g(c + 2, slot)

            return ()

        lax.fori_loop(0, npairs, body, ())
        if NC2 % 2:
            last = NC2 - 1
            w_copy(last - 2, 0).wait()
            start_g(last, 0)
            wait_g(last, 0)
            vsum(0)
            w_copy(last, 0).start()
            w_copy(last - 1, 1).wait()
            w_copy(last, 0).wait()
        else:
            w_copy(NC2 - 2, 0).wait()
            w_copy(NC2 - 1, 1).wait()

    return k(pa, pb, src, dst)


def _sc_segsum(h_edge, dst3, init, epw):
    nchunk = epw // SCHUNK

    @functools.partial(
        pl.kernel,
        out_type=jax.ShapeDtypeStruct((2 * N_NODES, D), jnp.float32),
        mesh=_sc_mesh(),
        scratch_types=[
            pltpu.VMEM((nchunk, SCHUNK), jnp.int32),
            pltpu.VMEM((SCHUNK, D), jnp.float32),
            pltpu.VMEM((SCHUNK, D), jnp.float32),
            pltpu.VMEM_SHARED((N_NODES, D), jnp.float32),
            pltpu.SemaphoreType.DMA,
            pltpu.SemaphoreType.DMA,
        ],
    )
    def k(he_hbm, dst_hbm, init_hbm, out_hbm, idx_v, r0, r1, shared,
          sl0, sl1):
        c = lax.axis_index("c")
        s = lax.axis_index("s")
        wid = c * SC_SUBCORES + s
        base = wid * epw

        @pl.when(s == 0)
        def _load_init():
            pltpu.sync_copy(init_hbm.at[pl.ds(c * N_NODES, N_NODES)], shared)

        pltpu.sync_copy(dst_hbm.at[wid], idx_v)
        plsc.subcore_barrier()

        slots = ((r0, sl0), (r1, sl1))

        def load_copy(chunk, slot):
            rv, sl = slots[slot]
            off = pl.multiple_of(base + chunk * SCHUNK, 8)
            return pltpu.make_async_copy(he_hbm.at[pl.ds(off, SCHUNK)], rv, sl)

        def process(chunk, slot):
            rv, _ = slots[slot]
            load_copy(chunk, slot).wait()
            pltpu.sync_copy(rv, shared.at[idx_v.at[chunk]], add=True)

        load_copy(0, 0).start()
        load_copy(1, 1).start()

        def body(j, _):
            a = 2 * j
            for slot in (0, 1):
                process(a + slot, slot)

            @pl.when(j < nchunk // 2 - 1)
            def _next():
                for slot in (0, 1):
                    load_copy(a + slot + 2, slot).start()

            return ()

        lax.fori_loop(0, nchunk // 2, body, ())
        if nchunk % 2:
            load_copy(nchunk - 1, 0).start()
            process(nchunk - 1, 0)

        plsc.subcore_barrier()

        rpt = 624
        r0 = s * rpt
        pltpu.sync_copy(shared.at[pl.ds(r0, rpt)],
                        out_hbm.at[pl.ds(c * N_NODES + r0, rpt)])

        @pl.when(s == 0)
        def _tail():
            t0 = SC_SUBCORES * rpt
            pltpu.sync_copy(shared.at[pl.ds(t0, N_NODES - t0)],
                            out_hbm.at[pl.ds(c * N_NODES + t0, N_NODES - t0)])

    return k(h_edge, dst3, init)


def _gslice(x, g):
    return x[GROUP_STARTS[g]:GROUP_STARTS[g] + GROUP_SIZES[g]]


def kernel(h_node, edge_index, h_edge, ew1, eb1, ew2, eb2, eln_s, eln_b,
           nw1, nb1, nw2, nb2, nln_s, nln_b):
    num_convs = ew1.shape[0]
    src = edge_index[0].astype(jnp.int32)
    dst = edge_index[1].astype(jnp.int32)
    src_g = [_gslice(src, g) for g in range(NSPLIT)]
    dst_g = [_gslice(dst, g) for g in range(NSPLIT)]
    dst3_g = [dst_g[g].reshape(NW, GROUP_EPW[g] // SCHUNK, SCHUNK)
              for g in range(NSPLIT)]
    he_g = [_gslice(h_edge, g) for g in range(NSPLIT)]
    zeros2 = jnp.zeros((2 * N_NODES, D), jnp.float32)

    r1 = lambda v: v.reshape(1, D)

    pa = pb = None
    for i in range(num_convs):
        cw = ew1[i, 2 * D:]
        if pa is None:
            pa, pb = _precompute(h_node, ew1[i, :D], ew1[i, D:2 * D])
        gg = [_sc_gather(pa, pb, src_g[g], dst_g[g], GROUP_EPW[g])
              for g in range(NSPLIT)]
        he_g = [_edge_mlp(gg[g], he_g[g], cw, ew2[i],
                          r1(eb1[i]), r1(eb2[i]), r1(eln_s[i]), r1(eln_b[i]))
                for g in range(NSPLIT)]
        parts = zeros2
        for g in range(NSPLIT):
            parts = _sc_segsum(he_g[g], dst3_g[g], parts, GROUP_EPW[g])
        if i + 1 < num_convs:
            h_node, pa, pb = _node_mlp(
                h_node, parts, nw1[i, :D], nw1[i, D:], nw2[i],
                r1(nb1[i]), r1(nb2[i]), r1(nln_s[i]), r1(nln_b[i]),
                next_a=ew1[i + 1, :D], next_b=ew1[i + 1, D:2 * D])
        else:
            h_node = _node_mlp(
                h_node, parts, nw1[i, :D], nw1[i, D:], nw2[i],
                r1(nb1[i]), r1(nb2[i]), r1(nln_s[i]), r1(nln_b[i]))
    return (h_node, jnp.concatenate(he_g, axis=0))
